# Initial kernel scaffold; baseline (speedup 1.0000x reference)
import jax
import jax.numpy as jnp
from jax.experimental import pallas as pl


def kernel(scores):
    raise NotImplementedError("write your pallas kernel here")


# speedup vs baseline: 11.9118x; 11.9118x over previous
import functools

import jax
import jax.numpy as jnp
import numpy as np
from jax import lax
from jax.experimental import pallas as pl
from jax.experimental.pallas import tpu as pltpu
from jax.experimental.pallas import tpu_sc as plsc

EPS = float(np.finfo(np.float32).tiny)
K_SEL = 8
N_IN = 1000000
NUM_SUBCORES = 16
LANES = 16
CHUNK = 62592
N_PAD = NUM_SUBCORES * CHUNK
UNROLL = 8

_MESH = plsc.VectorSubcoreMesh(
    core_axis_name="c", subcore_axis_name="s", num_cores=1
)


def _subset_kernel(scores_hbm, out_hbm, w_v, k_v, stage_v, stage_i, all_v,
                   all_i, sh_v, sh_i):
    sid = lax.axis_index("s")
    lane_iota = lax.iota(jnp.int32, LANES)
    zeros16 = jnp.zeros((LANES,), jnp.float32)

    def allreduce_sum(vec):
        stage_v[...] = vec
        pltpu.sync_copy(stage_v, sh_v.at[pl.ds(sid * LANES, LANES)])
        plsc.subcore_barrier()
        pltpu.sync_copy(sh_v, all_v)
        plsc.subcore_barrier()
        tot = zeros16
        for t in range(NUM_SUBCORES):
            tot = tot + all_v[pl.ds(t * LANES, LANES)]
        return jnp.sum(tot)

    pltpu.sync_copy(scores_hbm.at[pl.ds(sid * CHUNK, CHUNK)], w_v)

    @plsc.parallel_loop(0, CHUNK, step=LANES, unroll=UNROLL, carry=zeros16)
    def _(off, acc):
        e = jnp.exp(w_v[pl.ds(off, LANES)])
        w_v[pl.ds(off, LANES)] = e
        k_v[pl.ds(off, LANES)] = zeros16
        return acc + e

    z = allreduce_sum(_)

    for it in range(K_SEL):
        rzv = 1.0 / lax.broadcast(z, (LANES,))
        last = it == K_SEL - 1

        @plsc.parallel_loop(0, CHUNK, step=LANES, unroll=UNROLL, carry=zeros16)
        def _(off, acc):
            wv = w_v[pl.ds(off, LANES)]
            p = wv * rzv
            k_v[pl.ds(off, LANES)] = k_v[pl.ds(off, LANES)] + p
            if last:
                return acc
            wn = wv * jnp.maximum(1.0 - p, EPS)
            w_v[pl.ds(off, LANES)] = wn
            return acc + wn

        if not last:
            z = allreduce_sum(_)

    sel_vals = []
    sel_gidx = []
    for _r in range(K_SEL):
        init = (jnp.full((LANES,), -2.0, jnp.float32),
                jnp.zeros((LANES,), jnp.int32))

        @plsc.parallel_loop(0, CHUNK, step=LANES, unroll=UNROLL, carry=init)
        def _(off, c):
            bv, bi = c
            kv = k_v[pl.ds(off, LANES)]
            idx = lane_iota + off
            m = kv > bv
            return jnp.where(m, kv, bv), jnp.where(m, idx, bi)

        bv, bi = _
        stage_v[...] = bv
        stage_i[...] = bi
        pltpu.sync_copy(stage_v, sh_v.at[pl.ds(sid * LANES, LANES)])
        pltpu.sync_copy(stage_i, sh_i.at[pl.ds(sid * LANES, LANES)])
        plsc.subcore_barrier()
        pltpu.sync_copy(sh_v, all_v)
        pltpu.sync_copy(sh_i, all_i)
        plsc.subcore_barrier()

        bestv = jnp.float32(-2.0)
        bestg = jnp.int32(0)
        for t in range(NUM_SUBCORES):
            row_v = all_v[pl.ds(t * LANES, LANES)]
            row_i = all_i[pl.ds(t * LANES, LANES)]
            m = jnp.max(row_v)
            cand = jnp.where(row_v == m, row_i, jnp.int32(2**30))
            li = jnp.min(cand)
            g = li + t * CHUNK
            better = m > bestv
            bestv = jnp.where(better, m, bestv)
            bestg = jnp.where(better, g, bestg)
        sel_vals.append(bestv)
        sel_gidx.append(bestg)

        lo = bestg - sid * CHUNK
        is_owner = (lo >= 0) & (lo < CHUNK)

        @pl.when(is_owner)
        def _():
            lane = lo & (LANES - 1)
            base = lo - lane
            kv = k_v[pl.ds(base, LANES)]
            k_v[pl.ds(base, LANES)] = jnp.where(lane_iota == lane, -1.0, kv)

    @plsc.parallel_loop(0, CHUNK, step=LANES, unroll=UNROLL)
    def _(off):
        k_v[pl.ds(off, LANES)] = zeros16

    res_vec = zeros16
    lo_vec = jnp.zeros((LANES,), jnp.int32)
    own_vec = jnp.zeros((LANES,), jnp.int32)
    for r in range(K_SEL):
        lo = sel_gidx[r] - sid * CHUNK
        own = (lo >= 0) & (lo < CHUNK)
        rv = (1.0 - sel_vals[r]) + sel_vals[r]
        here = lane_iota == r
        res_vec = jnp.where(here, rv, res_vec)
        lo_vec = jnp.where(here, jnp.where(own, lo, 0), lo_vec)
        own_vec = jnp.where(here, jnp.where(own, 1, 0), own_vec)
    plsc.store_scatter(k_v, [lo_vec], res_vec, mask=own_vec > 0)

    pltpu.sync_copy(k_v, out_hbm.at[pl.ds(sid * CHUNK, CHUNK)])


@jax.jit
def kernel(scores):
    padded = jnp.concatenate(
        [scores, jnp.full((N_PAD - N_IN,), -1e30, jnp.float32)]
    )
    call = pl.kernel(
        _subset_kernel,
        out_type=jax.ShapeDtypeStruct((N_PAD,), jnp.float32),
        mesh=_MESH,
        compiler_params=pltpu.CompilerParams(needs_layout_passes=False),
        scratch_types=[
            pltpu.VMEM((CHUNK,), jnp.float32),
            pltpu.VMEM((CHUNK,), jnp.float32),
            pltpu.VMEM((LANES,), jnp.float32),
            pltpu.VMEM((LANES,), jnp.int32),
            pltpu.VMEM((NUM_SUBCORES * LANES,), jnp.float32),
            pltpu.VMEM((NUM_SUBCORES * LANES,), jnp.int32),
            pltpu.VMEM_SHARED((NUM_SUBCORES * LANES,), jnp.float32),
            pltpu.VMEM_SHARED((NUM_SUBCORES * LANES,), jnp.int32),
        ],
    )
    out = call(padded)
    return out[:N_IN]


# speedup vs baseline: 12.1394x; 1.0191x over previous
import functools

import jax
import jax.numpy as jnp
import numpy as np
from jax import lax
from jax.experimental import pallas as pl
from jax.experimental.pallas import tpu as pltpu
from jax.experimental.pallas import tpu_sc as plsc

EPS = float(np.finfo(np.float32).tiny)
K_SEL = 8
N_IN = 1000000
NUM_SUBCORES = 16
LANES = 16
CHUNK = 62592
N_PAD = NUM_SUBCORES * CHUNK
UNROLL = 8

_MESH = plsc.VectorSubcoreMesh(
    core_axis_name="c", subcore_axis_name="s", num_cores=1
)


def _subset_kernel(scores_hbm, out_hbm, w_v, k_v, stage_v, stage_i, all_v,
                   all_i, sh_v, sh_i):
    sid = lax.axis_index("s")
    lane_iota = lax.iota(jnp.int32, LANES)
    zeros16 = jnp.zeros((LANES,), jnp.float32)

    def allreduce_sum(vec):
        stage_v[...] = vec
        pltpu.sync_copy(stage_v, sh_v.at[pl.ds(sid * LANES, LANES)])
        plsc.subcore_barrier()
        pltpu.sync_copy(sh_v, all_v)
        plsc.subcore_barrier()
        tot = zeros16
        for t in range(NUM_SUBCORES):
            tot = tot + all_v[pl.ds(t * LANES, LANES)]
        return jnp.sum(tot)

    pltpu.sync_copy(scores_hbm.at[pl.ds(sid * CHUNK, CHUNK)], w_v)

    @plsc.parallel_loop(0, CHUNK, step=LANES, unroll=UNROLL, carry=zeros16)
    def _(off, acc):
        e = jnp.exp(w_v[pl.ds(off, LANES)])
        w_v[pl.ds(off, LANES)] = e
        k_v[pl.ds(off, LANES)] = zeros16
        return acc + e

    z = allreduce_sum(_)

    for it in range(K_SEL):
        rzv = 1.0 / lax.broadcast(z, (LANES,))
        last = it == K_SEL - 1

        @plsc.parallel_loop(0, CHUNK, step=LANES, unroll=UNROLL, carry=zeros16)
        def _(off, acc):
            wv = w_v[pl.ds(off, LANES)]
            p = wv * rzv
            k_v[pl.ds(off, LANES)] = k_v[pl.ds(off, LANES)] + p
            if last:
                w_v[pl.ds(off, LANES)] = zeros16
                return acc
            wn = wv * jnp.maximum(1.0 - p, EPS)
            w_v[pl.ds(off, LANES)] = wn
            return acc + wn

        if not last:
            z = allreduce_sum(_)

    sel_vals = []
    sel_gidx = []
    for _r in range(K_SEL):
        init = (jnp.full((LANES,), -2.0, jnp.float32),
                jnp.zeros((LANES,), jnp.int32))

        @plsc.parallel_loop(0, CHUNK, step=LANES, unroll=UNROLL, carry=init)
        def _(off, c):
            bv, bi = c
            kv = k_v[pl.ds(off, LANES)]
            idx = lane_iota + off
            m = kv > bv
            return jnp.where(m, kv, bv), jnp.where(m, idx, bi)

        bv, bi = _
        stage_v[...] = bv
        stage_i[...] = bi
        pltpu.sync_copy(stage_v, sh_v.at[pl.ds(sid * LANES, LANES)])
        pltpu.sync_copy(stage_i, sh_i.at[pl.ds(sid * LANES, LANES)])
        plsc.subcore_barrier()
        pltpu.sync_copy(sh_v, all_v)
        pltpu.sync_copy(sh_i, all_i)
        plsc.subcore_barrier()

        bestv = jnp.float32(-2.0)
        bestg = jnp.int32(0)
        for t in range(NUM_SUBCORES):
            row_v = all_v[pl.ds(t * LANES, LANES)]
            row_i = all_i[pl.ds(t * LANES, LANES)]
            m = jnp.max(row_v)
            cand = jnp.where(row_v == m, row_i, jnp.int32(2**30))
            li = jnp.min(cand)
            g = li + t * CHUNK
            better = m > bestv
            bestv = jnp.where(better, m, bestv)
            bestg = jnp.where(better, g, bestg)
        sel_vals.append(bestv)
        sel_gidx.append(bestg)

        lo = bestg - sid * CHUNK
        is_owner = (lo >= 0) & (lo < CHUNK)

        @pl.when(is_owner)
        def _():
            lane = lo & (LANES - 1)
            base = lo - lane
            kv = k_v[pl.ds(base, LANES)]
            k_v[pl.ds(base, LANES)] = jnp.where(lane_iota == lane, -1.0, kv)

    res_vec = zeros16
    lo_vec = jnp.zeros((LANES,), jnp.int32)
    own_vec = jnp.zeros((LANES,), jnp.int32)
    for r in range(K_SEL):
        lo = sel_gidx[r] - sid * CHUNK
        own = (lo >= 0) & (lo < CHUNK)
        rv = (1.0 - sel_vals[r]) + sel_vals[r]
        here = lane_iota == r
        res_vec = jnp.where(here, rv, res_vec)
        lo_vec = jnp.where(here, jnp.where(own, lo, 0), lo_vec)
        own_vec = jnp.where(here, jnp.where(own, 1, 0), own_vec)
    plsc.store_scatter(w_v, [lo_vec], res_vec, mask=own_vec > 0)

    pltpu.sync_copy(w_v, out_hbm.at[pl.ds(sid * CHUNK, CHUNK)])


@jax.jit
def kernel(scores):
    padded = jnp.concatenate(
        [scores, jnp.full((N_PAD - N_IN,), -1e30, jnp.float32)]
    )
    call = pl.kernel(
        _subset_kernel,
        out_type=jax.ShapeDtypeStruct((N_PAD,), jnp.float32),
        mesh=_MESH,
        compiler_params=pltpu.CompilerParams(needs_layout_passes=False),
        scratch_types=[
            pltpu.VMEM((CHUNK,), jnp.float32),
            pltpu.VMEM((CHUNK,), jnp.float32),
            pltpu.VMEM((LANES,), jnp.float32),
            pltpu.VMEM((LANES,), jnp.int32),
            pltpu.VMEM((NUM_SUBCORES * LANES,), jnp.float32),
            pltpu.VMEM((NUM_SUBCORES * LANES,), jnp.int32),
            pltpu.VMEM_SHARED((NUM_SUBCORES * LANES,), jnp.float32),
            pltpu.VMEM_SHARED((NUM_SUBCORES * LANES,), jnp.int32),
        ],
    )
    out = call(padded)
    return out[:N_IN]


# speedup vs baseline: 17.4128x; 1.4344x over previous
import functools

import jax
import jax.numpy as jnp
import numpy as np
from jax import lax
from jax.experimental import pallas as pl
from jax.experimental.pallas import tpu as pltpu
from jax.experimental.pallas import tpu_sc as plsc

EPS = float(np.finfo(np.float32).tiny)
K_SEL = 8
N_IN = 1000000
NUM_SUBCORES = 16
LANES = 16
CHUNK = 62592
N_PAD = NUM_SUBCORES * CHUNK
UNROLL = 8

_MESH = plsc.VectorSubcoreMesh(
    core_axis_name="c", subcore_axis_name="s", num_cores=1
)


def _subset_kernel(scores_hbm, out_hbm, w_v, k_v, stage_v, stage_i, all_v,
                   all_i, sh_v, sh_i):
    sid = lax.axis_index("s")
    lane_iota = lax.iota(jnp.int32, LANES)
    zeros16 = jnp.zeros((LANES,), jnp.float32)

    def allreduce_sum(vec):
        stage_v[...] = vec
        pltpu.sync_copy(stage_v, sh_v.at[pl.ds(sid * LANES, LANES)])
        plsc.subcore_barrier()
        pltpu.sync_copy(sh_v, all_v)
        plsc.subcore_barrier()
        tot = zeros16
        for t in range(NUM_SUBCORES):
            tot = tot + all_v[pl.ds(t * LANES, LANES)]
        return jnp.sum(tot)

    pltpu.sync_copy(scores_hbm.at[pl.ds(sid * CHUNK, CHUNK)], w_v)

    @plsc.parallel_loop(0, CHUNK, step=LANES, unroll=UNROLL, carry=zeros16)
    def _(off, acc):
        e = jnp.exp(w_v[pl.ds(off, LANES)])
        w_v[pl.ds(off, LANES)] = e
        k_v[pl.ds(off, LANES)] = zeros16
        return acc + e

    z = allreduce_sum(_)

    for it in range(K_SEL - 1):
        rzv = 1.0 / lax.broadcast(z, (LANES,))

        @plsc.parallel_loop(0, CHUNK, step=LANES, unroll=UNROLL, carry=zeros16)
        def _(off, acc):
            wv = w_v[pl.ds(off, LANES)]
            p = wv * rzv
            k_v[pl.ds(off, LANES)] = k_v[pl.ds(off, LANES)] + p
            wn = wv * jnp.maximum(1.0 - p, EPS)
            w_v[pl.ds(off, LANES)] = wn
            return acc + wn

        z = allreduce_sum(_)

    rzv = 1.0 / lax.broadcast(z, (LANES,))
    lastinit = (jnp.full((LANES,), -2.0, jnp.float32),
                jnp.zeros((LANES,), jnp.int32))

    @plsc.parallel_loop(0, CHUNK, step=LANES, unroll=UNROLL, carry=lastinit)
    def _(off, c):
        bv, bi = c
        knew = k_v[pl.ds(off, LANES)] + w_v[pl.ds(off, LANES)] * rzv
        k_v[pl.ds(off, LANES)] = knew
        w_v[pl.ds(off, LANES)] = zeros16
        m = knew > bv
        return jnp.where(m, knew, bv), jnp.where(m, lane_iota + off, bi)

    bv, bi = _

    stage_v[...] = bv
    stage_i[...] = bi + sid * CHUNK
    pltpu.sync_copy(stage_v, sh_v.at[pl.ds(sid * LANES, LANES)])
    pltpu.sync_copy(stage_i, sh_i.at[pl.ds(sid * LANES, LANES)])
    plsc.subcore_barrier()
    pltpu.sync_copy(sh_v, all_v)
    pltpu.sync_copy(sh_i, all_i)
    plsc.subcore_barrier()

    big_i = jnp.int32(2**30)
    cand_v = zeros16
    cand_g = jnp.zeros((LANES,), jnp.int32)
    tau = jnp.float32(0.0)
    for r in range(K_SEL):
        tv = jnp.full((LANES,), -2.0, jnp.float32)
        tg = jnp.full((LANES,), 0, jnp.int32)
        for t in range(NUM_SUBCORES):
            rv = all_v[pl.ds(t * LANES, LANES)]
            rg = all_i[pl.ds(t * LANES, LANES)]
            m = rv > tv
            tv = jnp.where(m, rv, tv)
            tg = jnp.where(m, rg, tg)
        m = jnp.max(tv)
        g = jnp.min(jnp.where(tv == m, tg, big_i))
        here = lane_iota == r
        cand_v = jnp.where(here, m, cand_v)
        cand_g = jnp.where(here, g, cand_g)
        tau = m
        for t in range(NUM_SUBCORES):
            rv = all_v[pl.ds(t * LANES, LANES)]
            rg = all_i[pl.ds(t * LANES, LANES)]
            all_v[pl.ds(t * LANES, LANES)] = jnp.where(rg == g, -2.0, rv)

    tauv = lax.broadcast(tau, (LANES,))

    @plsc.parallel_loop(0, CHUNK, step=LANES, unroll=UNROLL, carry=zeros16)
    def _(off, acc):
        return acc + jnp.where(k_v[pl.ds(off, LANES)] >= tauv, 1.0, 0.0)

    cnt = allreduce_sum(_)

    stage_v[...] = cand_v
    stage_i[...] = cand_g

    @pl.when(cnt != 8.0)
    def _():
        for r in range(K_SEL):
            init = (jnp.full((LANES,), -2.0, jnp.float32),
                    jnp.zeros((LANES,), jnp.int32))

            @plsc.parallel_loop(0, CHUNK, step=LANES, unroll=UNROLL,
                                carry=init)
            def _(off, c):
                fv, fi = c
                kv = k_v[pl.ds(off, LANES)]
                m = kv > fv
                return (jnp.where(m, kv, fv),
                        jnp.where(m, lane_iota + off, fi))

            fv, fi = _
            sc_v = stage_v[...]
            sc_i = stage_i[...]
            stage_v[...] = fv
            stage_i[...] = fi + sid * CHUNK
            pltpu.sync_copy(stage_v, sh_v.at[pl.ds(sid * LANES, LANES)])
            pltpu.sync_copy(stage_i, sh_i.at[pl.ds(sid * LANES, LANES)])
            plsc.subcore_barrier()
            pltpu.sync_copy(sh_v, all_v)
            pltpu.sync_copy(sh_i, all_i)
            plsc.subcore_barrier()

            tv = jnp.full((LANES,), -2.0, jnp.float32)
            tg = jnp.full((LANES,), 0, jnp.int32)
            for t in range(NUM_SUBCORES):
                rv = all_v[pl.ds(t * LANES, LANES)]
                rg = all_i[pl.ds(t * LANES, LANES)]
                m = rv > tv
                tv = jnp.where(m, rv, tv)
                tg = jnp.where(m, rg, tg)
            m = jnp.max(tv)
            g = jnp.min(jnp.where(tv == m, tg, big_i))
            here = lane_iota == r
            stage_v[...] = jnp.where(here, m, sc_v)
            stage_i[...] = jnp.where(here, g, sc_i)

            lo = g - sid * CHUNK
            is_owner = (lo >= 0) & (lo < CHUNK)

            @pl.when(is_owner)
            def _():
                lane = lo & (LANES - 1)
                base = lo - lane
                kv = k_v[pl.ds(base, LANES)]
                k_v[pl.ds(base, LANES)] = jnp.where(
                    lane_iota == lane, -1.0, kv)

    val_vec = stage_v[...]
    g_vec = stage_i[...]
    res_vec = (1.0 - val_vec) + val_vec
    lo_vec = g_vec - sid * CHUNK
    own = (lo_vec >= 0) & (lo_vec < CHUNK) & (lane_iota < K_SEL)
    safe_lo = jnp.where(own, lo_vec, 0)
    plsc.store_scatter(w_v, [safe_lo], res_vec, mask=own)

    pltpu.sync_copy(w_v, out_hbm.at[pl.ds(sid * CHUNK, CHUNK)])


@jax.jit
def kernel(scores):
    padded = jnp.concatenate(
        [scores, jnp.full((N_PAD - N_IN,), -1e30, jnp.float32)]
    )
    call = pl.kernel(
        _subset_kernel,
        out_type=jax.ShapeDtypeStruct((N_PAD,), jnp.float32),
        mesh=_MESH,
        compiler_params=pltpu.CompilerParams(needs_layout_passes=False),
        scratch_types=[
            pltpu.VMEM((CHUNK,), jnp.float32),
            pltpu.VMEM((CHUNK,), jnp.float32),
            pltpu.VMEM((LANES,), jnp.float32),
            pltpu.VMEM((LANES,), jnp.int32),
            pltpu.VMEM((NUM_SUBCORES * LANES,), jnp.float32),
            pltpu.VMEM((NUM_SUBCORES * LANES,), jnp.int32),
            pltpu.VMEM_SHARED((NUM_SUBCORES * LANES,), jnp.float32),
            pltpu.VMEM_SHARED((NUM_SUBCORES * LANES,), jnp.int32),
        ],
    )
    out = call(padded)
    return out[:N_IN]
